# Initial kernel scaffold; baseline (speedup 1.0000x reference)
import jax
import jax.numpy as jnp
from jax.experimental import pallas as pl


def kernel(subst_nodes, adj_nodes, rel_positions, subst_table, adj_table, rel_table, W, b):
    raise NotImplementedError("write your pallas kernel here")


# speedup vs baseline: 1.3692x; 1.3692x over previous
import functools

import jax
import jax.numpy as jnp
from jax import lax
from jax.experimental import pallas as pl
from jax.experimental.pallas import tpu as pltpu
from jax.experimental.pallas import tpu_sc as plsc

D_MODEL = 2048
D4 = D_MODEL // 4
N_COMBO = 44
NB = 4
CB = D_MODEL // NB

L = 16
K = 16


def _ctable_body(st_ref, at_ref, rt_ref, w_ref, b_ref, out_ref):
    w = w_ref[...]
    ps = jnp.dot(st_ref[...], w[0:D4, :], preferred_element_type=jnp.float32)
    pa = jnp.dot(at_ref[...], w[D4:2 * D4, :], preferred_element_type=jnp.float32)
    pr = jnp.dot(rt_ref[...], w[2 * D4:3 * D4, :], preferred_element_type=jnp.float32)
    bv = b_ref[...]
    for s in range(2):
        for a in range(2):
            out_ref[s * 2 + a, :, :] = pr + (ps[s] + pa[a] + bv)[None, :]


def _build_ctable(subst_table, adj_table, rel_table, W, b):
    c4 = pl.pallas_call(
        _ctable_body,
        grid=(NB,),
        in_specs=[
            pl.BlockSpec((2, D4), lambda c: (0, 0)),
            pl.BlockSpec((2, D4), lambda c: (0, 0)),
            pl.BlockSpec((11, D4), lambda c: (0, 0)),
            pl.BlockSpec((3 * D4, CB), lambda c: (0, c)),
            pl.BlockSpec((CB,), lambda c: (c,)),
        ],
        out_specs=pl.BlockSpec((4, 11, CB), lambda c: (0, 0, c)),
        out_shape=jax.ShapeDtypeStruct((4, 11, D_MODEL), jnp.float32),
    )(subst_table, adj_table, rel_table, W, b)
    return c4.reshape(N_COMBO, D_MODEL)


def _make_sc_lookup(ntok):
    info = plsc.get_sparse_core_info()
    nw = info.num_cores * info.num_subcores
    tpw = ntok // nw
    nch = tpw // K
    mesh = plsc.VectorSubcoreMesh(core_axis_name="c", subcore_axis_name="s")

    @functools.partial(
        pl.kernel,
        out_type=jax.ShapeDtypeStruct((ntok, D_MODEL), jnp.float32),
        mesh=mesh,
        scratch_types=[
            pltpu.VMEM((tpw,), jnp.int32),
            pltpu.VMEM((tpw,), jnp.int32),
            pltpu.VMEM((tpw,), jnp.int32),
            pltpu.VMEM((nch, K), jnp.int32),
            pltpu.VMEM((2, K, D_MODEL), jnp.float32),
            pltpu.SemaphoreType.DMA,
        ],
    )
    def sc_lookup(sub_hbm, adj_hbm, rel_hbm, c_hbm, out_hbm,
                  sv, av, rv, idx2, rows, sem):
        wid = lax.axis_index("s") * info.num_cores + lax.axis_index("c")
        base = wid * tpw
        pltpu.sync_copy(sub_hbm.at[pl.ds(base, tpw)], sv)
        pltpu.sync_copy(adj_hbm.at[pl.ds(base, tpw)], av)
        pltpu.sync_copy(rel_hbm.at[pl.ds(base, tpw)], rv)

        for i in range(tpw // L):
            s = sv[pl.ds(i * L, L)]
            a = av[pl.ds(i * L, L)]
            r = rv[pl.ds(i * L, L)]
            c = s * 22 + a * 11 + jnp.clip(r + 5, 0, 10)
            idx2[i] = jnp.clip(c, 0, N_COMBO - 1)

        def outer(jo, carry):
            for bsub in range(2):
                pltpu.async_copy(c_hbm.at[idx2.at[2 * jo + bsub]],
                                 rows.at[bsub], sem)
            for bsub in range(2):
                pltpu.make_async_copy(c_hbm.at[idx2.at[2 * jo + bsub]],
                                      rows.at[bsub], sem).wait()
            for bsub in range(2):
                j = 2 * jo + bsub
                pltpu.sync_copy(rows.at[bsub],
                                out_hbm.at[pl.ds(base + j * K, K)])
            return carry

        lax.fori_loop(0, nch // 2, outer, 0)

    return sc_lookup


def kernel(subst_nodes, adj_nodes, rel_positions, subst_table, adj_table,
           rel_table, W, b):
    bdim, sdim = subst_nodes.shape
    ntok = bdim * sdim
    ctable = _build_ctable(subst_table, adj_table, rel_table, W, b)
    sc_lookup = _make_sc_lookup(ntok)
    out = sc_lookup(
        subst_nodes.reshape(ntok).astype(jnp.int32),
        adj_nodes.reshape(ntok).astype(jnp.int32),
        rel_positions.reshape(ntok).astype(jnp.int32),
        ctable,
    )
    return out.reshape(bdim, sdim, D_MODEL)
